# Initial kernel scaffold; baseline (speedup 1.0000x reference)
import jax
import jax.numpy as jnp
from jax.experimental import pallas as pl


def kernel(f_atoms, f_bonds, a2b, b2a, b2revb, atom_segments, W_i, W_h, W_o, b_o):
    raise NotImplementedError("write your pallas kernel here")


# speedup vs baseline: 2.1601x; 2.1601x over previous
import functools

import jax
import jax.numpy as jnp
from jax import lax
from jax.experimental import pallas as pl
from jax.experimental.pallas import tpu as pltpu
from jax.experimental.pallas import tpu_sc as plsc

N_ATOMS = 10000
N_BONDS = 320000
ATOM_FDIM = 128
BOND_FDIM = 144
H = 128
MAX_NB = 32
N_MOLS = 1000

NC = 2
NS = 16
NW = NC * NS

B_CH = 79
NBP = NW * B_CH * 128
A_CH = 80
NAP = NW * A_CH * 4
R_CH = 3
NRP = NW * R_CH * 128
NMP = 1024
NB_GRID = 626
NA_GRID = 20

_f32 = jnp.float32


def _mm_body(x_ref, w_ref, o_ref):
    o_ref[...] = jnp.dot(x_ref[...], w_ref[...], preferred_element_type=_f32)


def _proj_bonds(f_bonds, w_t):
    return pl.pallas_call(
        _mm_body,
        grid=(NB_GRID,),
        in_specs=[
            pl.BlockSpec((512, BOND_FDIM), lambda i: (i, 0)),
            pl.BlockSpec((BOND_FDIM, H), lambda i: (0, 0)),
        ],
        out_specs=pl.BlockSpec((512, H), lambda i: (i, 0)),
        out_shape=jax.ShapeDtypeStruct((NBP, H), _f32),
    )(f_bonds, w_t)


def _update_body(inp_ref, t_ref, w_ref, o_ref):
    acc = jnp.dot(t_ref[...], w_ref[...], preferred_element_type=_f32)
    o_ref[...] = jnp.maximum(inp_ref[...] + acc, 0.0)


def _update(inp, t, wh_t):
    return pl.pallas_call(
        _update_body,
        grid=(NB_GRID,),
        in_specs=[
            pl.BlockSpec((512, H), lambda i: (i, 0)),
            pl.BlockSpec((512, H), lambda i: (i, 0)),
            pl.BlockSpec((H, H), lambda i: (0, 0)),
        ],
        out_specs=pl.BlockSpec((512, H), lambda i: (i, 0)),
        out_shape=jax.ShapeDtypeStruct((NBP, H), _f32),
    )(inp, t, wh_t)


def _head_body(fa_ref, am_ref, wa_ref, wh_ref, b_ref, o_ref):
    acc = jnp.dot(fa_ref[...], wa_ref[...], preferred_element_type=_f32)
    acc = acc + jnp.dot(am_ref[...], wh_ref[...], preferred_element_type=_f32)
    o_ref[...] = jnp.maximum(acc + b_ref[...], 0.0)


def _head(f_atoms, a_msg, wa_t, wh_t, b_o):
    return pl.pallas_call(
        _head_body,
        grid=(NA_GRID,),
        in_specs=[
            pl.BlockSpec((512, ATOM_FDIM), lambda i: (i, 0)),
            pl.BlockSpec((512, H), lambda i: (i, 0)),
            pl.BlockSpec((ATOM_FDIM, H), lambda i: (0, 0)),
            pl.BlockSpec((H, H), lambda i: (0, 0)),
            pl.BlockSpec((1, H), lambda i: (0, 0)),
        ],
        out_specs=pl.BlockSpec((512, H), lambda i: (i, 0)),
        out_shape=jax.ShapeDtypeStruct((NRP, H), _f32),
    )(f_atoms, a_msg, wa_t, wh_t, b_o)


def _combine_body(s_ref, c_ref, o_ref):
    s = s_ref[0] + s_ref[1]
    c = c_ref[0] + c_ref[1]
    cnt = c[:N_MOLS, 0:1]
    o_ref[...] = s[:N_MOLS] / jnp.maximum(cnt, 1.0)


def _combine(partial_sums, partial_cnts):
    return pl.pallas_call(
        _combine_body,
        grid=(1,),
        in_specs=[
            pl.BlockSpec((2, NMP, H), lambda i: (0, 0, 0)),
            pl.BlockSpec((2, NMP, 128), lambda i: (0, 0, 0)),
        ],
        out_specs=pl.BlockSpec((N_MOLS, H), lambda i: (0, 0)),
        out_shape=jax.ShapeDtypeStruct((N_MOLS, H), _f32),
    )(partial_sums, partial_cnts)


_MESH = dict(core_axis_name="c", subcore_axis_name="s", num_cores=NC,
             num_subcores=NS)


def _gather_sum_body(msg_hbm, idx_hbm, out_hbm, idx_v, buf_v, out_v, sem,
                     *, relu):
    w = lax.axis_index("s") * NC + lax.axis_index("c")
    pltpu.sync_copy(idx_hbm.at[w], idx_v)

    def chunk(s, carry):
        pltpu.async_copy(msg_hbm.at[idx_v.at[s]], buf_v, sem).wait()
        for a in range(4):
            def nb(j, accs):
                r = a * MAX_NB + j
                vals = [buf_v[r, pl.ds(c * 16, 16)] for c in range(8)]
                if relu:
                    vals = [jnp.maximum(v, 0.0) for v in vals]
                return tuple(x + v for x, v in zip(accs, vals))

            accs = lax.fori_loop(
                0, MAX_NB, nb, tuple(jnp.zeros((16,), _f32) for _ in range(8)))
            row = s * 4 + a
            for c in range(8):
                out_v[row, pl.ds(c * 16, 16)] = accs[c]
        return carry

    lax.fori_loop(0, A_CH, chunk, 0)
    pltpu.sync_copy(out_v, out_hbm.at[pl.ds(w * (A_CH * 4), A_CH * 4)])


def _gather_sum(msg, idx, *, relu):
    body = functools.partial(_gather_sum_body, relu=relu)
    return pl.kernel(
        body,
        out_type=jax.ShapeDtypeStruct((NAP, H), _f32),
        mesh=plsc.VectorSubcoreMesh(**_MESH),
        scratch_types=[
            pltpu.VMEM((A_CH, 128), jnp.int32),
            pltpu.VMEM((128, H), _f32),
            pltpu.VMEM((A_CH * 4, H), _f32),
            pltpu.SemaphoreType.DMA,
        ],
    )(msg, idx)


def _edge_msg_body(amsg_hbm, msg_hbm, ia_hbm, ib_hbm, t_hbm,
                   ia_v, ib_v, bufa_v, bufb_v, out_v, sem, *, relu):
    w = lax.axis_index("s") * NC + lax.axis_index("c")
    pltpu.sync_copy(ia_hbm.at[w], ia_v)
    pltpu.sync_copy(ib_hbm.at[w], ib_v)

    def chunk(s, carry):
        ca = pltpu.async_copy(amsg_hbm.at[ia_v.at[s]], bufa_v, sem)
        cb = pltpu.async_copy(msg_hbm.at[ib_v.at[s]], bufb_v, sem)
        ca.wait()
        cb.wait()

        def row(r, carry2):
            for c in range(8):
                a = bufa_v[r, pl.ds(c * 16, 16)]
                b = bufb_v[r, pl.ds(c * 16, 16)]
                if relu:
                    b = jnp.maximum(b, 0.0)
                out_v[r, pl.ds(c * 16, 16)] = a - b
            return carry2

        lax.fori_loop(0, 128, row, 0)
        pltpu.sync_copy(out_v, t_hbm.at[pl.ds(w * (B_CH * 128) + s * 128, 128)])
        return carry

    lax.fori_loop(0, B_CH, chunk, 0)


def _edge_msg(a_msg, msg, ia, ib, *, relu):
    body = functools.partial(_edge_msg_body, relu=relu)
    return pl.kernel(
        body,
        out_type=jax.ShapeDtypeStruct((NBP, H), _f32),
        mesh=plsc.VectorSubcoreMesh(**_MESH),
        scratch_types=[
            pltpu.VMEM((B_CH, 128), jnp.int32),
            pltpu.VMEM((B_CH, 128), jnp.int32),
            pltpu.VMEM((128, H), _f32),
            pltpu.VMEM((128, H), _f32),
            pltpu.VMEM((128, H), _f32),
            pltpu.SemaphoreType.DMA,
        ],
    )(a_msg, msg, ia, ib)


def _readout_body(ah_hbm, seg_hbm, zs_hbm, zc_hbm, ones_hbm,
                  sums_hbm, cnts_hbm,
                  idx_v, rows_v, ones_v, acc_sh, cnt_sh, sem):
    cid = lax.axis_index("c")
    sid = lax.axis_index("s")
    w = sid * NC + cid

    @pl.when(sid == 0)
    def _init():
        pltpu.sync_copy(zs_hbm, acc_sh)
        pltpu.sync_copy(zc_hbm, cnt_sh)

    pltpu.sync_copy(ones_hbm, ones_v)
    pltpu.sync_copy(seg_hbm.at[w], idx_v)
    plsc.subcore_barrier()

    for ch in range(R_CH):
        base = w * (R_CH * 128) + ch * 128
        pltpu.async_copy(ah_hbm.at[pl.ds(base, 128)], rows_v, sem).wait()
        pltpu.sync_copy(rows_v, acc_sh.at[idx_v.at[ch]], add=True)
        pltpu.sync_copy(ones_v, cnt_sh.at[idx_v.at[ch]], add=True)

    plsc.subcore_barrier()

    @pl.when(sid == 0)
    def _flush():
        pltpu.sync_copy(acc_sh, sums_hbm.at[cid])
        pltpu.sync_copy(cnt_sh, cnts_hbm.at[cid])


def _readout(atom_hiddens, seg_idx):
    zs = jnp.zeros((NMP, H), _f32)
    zc = jnp.zeros((NMP, 128), _f32)
    ones = jnp.ones((128, 128), _f32)
    return pl.kernel(
        _readout_body,
        out_type=(
            jax.ShapeDtypeStruct((NC, NMP, H), _f32),
            jax.ShapeDtypeStruct((NC, NMP, 128), _f32),
        ),
        mesh=plsc.VectorSubcoreMesh(**_MESH),
        scratch_types=[
            pltpu.VMEM((R_CH, 128), jnp.int32),
            pltpu.VMEM((128, H), _f32),
            pltpu.VMEM((128, 128), _f32),
            pltpu.VMEM_SHARED((NMP, H), _f32),
            pltpu.VMEM_SHARED((NMP, 128), _f32),
            pltpu.SemaphoreType.DMA,
        ],
    )(atom_hiddens, seg_idx, zs, zc, ones)


def kernel(f_atoms, f_bonds, a2b, b2a, b2revb, atom_segments, W_i, W_h, W_o,
           b_o):
    na1 = N_ATOMS + 1
    nb1 = N_BONDS + 1

    pad_a = NAP - na1
    a2b_idx = jnp.concatenate(
        [a2b.reshape(-1),
         (jnp.arange(pad_a * MAX_NB, dtype=jnp.int32) * 7) % nb1]
    ).reshape(NW, A_CH, 128)

    pad_b = NBP - nb1
    ia = jnp.concatenate(
        [b2a, (jnp.arange(pad_b, dtype=jnp.int32) * 11) % na1]
    ).reshape(NW, B_CH, 128)
    ib = jnp.concatenate(
        [b2revb, (jnp.arange(pad_b, dtype=jnp.int32) * 13) % nb1]
    ).reshape(NW, B_CH, 128)

    pad_r = NRP - na1
    seg_idx = jnp.concatenate(
        [jnp.full((1,), N_MOLS, jnp.int32),
         atom_segments,
         N_MOLS + (jnp.arange(pad_r, dtype=jnp.int32) % (NMP - N_MOLS))]
    ).reshape(NW, R_CH, 128)

    wi_t = W_i.T
    wh_t = W_h.T
    woa_t = W_o[:, :ATOM_FDIM].T
    woh_t = W_o[:, ATOM_FDIM:].T
    b2 = b_o.reshape(1, H)

    inp = _proj_bonds(f_bonds, wi_t)

    am = _gather_sum(inp, a2b_idx, relu=True)
    t = _edge_msg(am, inp, ia, ib, relu=True)
    msg = _update(inp, t, wh_t)

    am = _gather_sum(msg, a2b_idx, relu=False)
    t = _edge_msg(am, msg, ia, ib, relu=False)
    msg = _update(inp, t, wh_t)

    am = _gather_sum(msg, a2b_idx, relu=False)
    ah = _head(f_atoms, am, woa_t, woh_t, b2)

    sums, cnts = _readout(ah, seg_idx)
    return _combine(sums, cnts)
